# Initial kernel scaffold; baseline (speedup 1.0000x reference)
import jax
import jax.numpy as jnp
from jax.experimental import pallas as pl


def kernel(src_indices, tgt_indices, source_table, target_table):
    raise NotImplementedError("write your pallas kernel here")


# speedup vs baseline: 4.3132x; 4.3132x over previous
import functools

import jax
import jax.numpy as jnp
from jax import lax
from jax.experimental import pallas as pl
from jax.experimental.pallas import tpu as pltpu
from jax.experimental.pallas import tpu_sc as plsc

VOCAB = 100000
EMBED = 64
BATCH = 4096
SEQ = 50
TOTAL = BATCH * SEQ

NC = 2
NS = 16
NW = NC * NS
RPW = TOTAL // NW
CHUNK = 128
NCH = RPW // CHUNK

_mesh = plsc.VectorSubcoreMesh(core_axis_name="c", subcore_axis_name="s")


@functools.partial(
    pl.kernel,
    out_type=(
        jax.ShapeDtypeStruct((TOTAL, EMBED), jnp.float32),
        jax.ShapeDtypeStruct((TOTAL, EMBED), jnp.float32),
    ),
    mesh=_mesh,
    compiler_params=pltpu.CompilerParams(use_tc_tiling_on_sc=False),
    scratch_types=[
        pltpu.VMEM((NCH, CHUNK), jnp.int32),
        pltpu.VMEM((CHUNK, EMBED), jnp.float32),
        pltpu.SemaphoreType.DMA,
    ],
)
def _emb_lookup(src_idx, tgt_idx, src_tab, tgt_tab, src_out, tgt_out,
                idx_v, rows_v, sem):
    wid = lax.axis_index("s") * NC + lax.axis_index("c")
    base = wid * RPW

    def one_table(idx_hbm, tab_hbm, out_hbm):
        pltpu.sync_copy(idx_hbm.at[wid], idx_v)

        def body(j, carry):
            pltpu.async_copy(tab_hbm.at[idx_v.at[j]], rows_v, sem).wait()
            pltpu.sync_copy(rows_v, out_hbm.at[pl.ds(base + j * CHUNK, CHUNK)])
            return carry

        lax.fori_loop(0, NCH, body, 0)

    one_table(src_idx, src_tab, src_out)
    one_table(tgt_idx, tgt_tab, tgt_out)


def kernel(src_indices, tgt_indices, source_table, target_table):
    si = src_indices.reshape(NW, NCH, CHUNK)
    ti = tgt_indices.reshape(NW, NCH, CHUNK)
    so, to = _emb_lookup(si, ti, source_table, target_table)
    return (so.reshape(BATCH, SEQ, EMBED), to.reshape(BATCH, SEQ, EMBED))


# speedup vs baseline: 4.9981x; 1.1588x over previous
import functools

import jax
import jax.numpy as jnp
from jax import lax
from jax.experimental import pallas as pl
from jax.experimental.pallas import tpu as pltpu
from jax.experimental.pallas import tpu_sc as plsc

VOCAB = 100000
EMBED = 64
BATCH = 4096
SEQ = 50
TOTAL = BATCH * SEQ

NC = 2
NS = 16
NW = NC * NS
RPW = TOTAL // NW
CHUNK = 128
NCH = RPW // CHUNK
K = 5
G = NCH // K

_mesh = plsc.VectorSubcoreMesh(core_axis_name="c", subcore_axis_name="s")


def _one_table(idx_v, tab_hbm, out_hbm, base_c, set_a, set_b,
               gs_a, gs_b, ws_a, ws_b):
    def issue_gathers(g, dst, gsem):
        for i in range(K):
            pltpu.async_copy(tab_hbm.at[idx_v.at[g * K + i]], dst.at[i], gsem)

    def drain_gathers(dst, gsem):
        for i in range(K):
            pltpu.make_async_copy(
                tab_hbm.at[pl.ds(0, CHUNK)], dst.at[i], gsem).wait()

    def issue_writeback(g, src, wsem):
        pltpu.async_copy(src, out_hbm.at[pl.ds(base_c + g * K, K)], wsem)

    def wait_writeback(src, wsem):
        pltpu.make_async_copy(
            src, out_hbm.at[pl.ds(base_c, K)], wsem).wait()

    issue_gathers(0, set_a, gs_a)

    def body(t, carry):
        @pl.when((t % 2) == 0)
        def _():
            @pl.when(t >= 1)
            def _():
                wait_writeback(set_b, ws_b)
            issue_gathers(t + 1, set_b, gs_b)
            drain_gathers(set_a, gs_a)
            issue_writeback(t, set_a, ws_a)

        @pl.when((t % 2) == 1)
        def _():
            wait_writeback(set_a, ws_a)
            issue_gathers(t + 1, set_a, gs_a)
            drain_gathers(set_b, gs_b)
            issue_writeback(t, set_b, ws_b)

        return carry

    lax.fori_loop(0, G - 1, body, 0)

    drain_gathers(set_b, gs_b)
    issue_writeback(G - 1, set_b, ws_b)
    wait_writeback(set_a, ws_a)
    wait_writeback(set_b, ws_b)


@functools.partial(
    pl.kernel,
    out_type=(
        jax.ShapeDtypeStruct((TOTAL // CHUNK, CHUNK, EMBED), jnp.float32),
        jax.ShapeDtypeStruct((TOTAL // CHUNK, CHUNK, EMBED), jnp.float32),
    ),
    mesh=_mesh,
    compiler_params=pltpu.CompilerParams(use_tc_tiling_on_sc=False),
    scratch_types=[
        pltpu.VMEM((NCH, CHUNK), jnp.int32),
        pltpu.VMEM((NCH, CHUNK), jnp.int32),
        pltpu.VMEM((K, CHUNK, EMBED), jnp.float32),
        pltpu.VMEM((K, CHUNK, EMBED), jnp.float32),
        pltpu.SemaphoreType.DMA,
        pltpu.SemaphoreType.DMA,
        pltpu.SemaphoreType.DMA,
        pltpu.SemaphoreType.DMA,
    ],
)
def _emb_lookup(src_idx, tgt_idx, src_tab, tgt_tab, src_out, tgt_out,
                idx_s, idx_t, set_a, set_b, gs_a, gs_b, ws_a, ws_b):
    wid = lax.axis_index("s") * NC + lax.axis_index("c")
    base_c = wid * NCH
    pltpu.sync_copy(src_idx.at[wid], idx_s)
    pltpu.sync_copy(tgt_idx.at[wid], idx_t)
    _one_table(idx_s, src_tab, src_out, base_c, set_a, set_b,
               gs_a, gs_b, ws_a, ws_b)
    _one_table(idx_t, tgt_tab, tgt_out, base_c, set_a, set_b,
               gs_a, gs_b, ws_a, ws_b)


def kernel(src_indices, tgt_indices, source_table, target_table):
    si = src_indices.reshape(NW, NCH, CHUNK)
    ti = tgt_indices.reshape(NW, NCH, CHUNK)
    so, to = _emb_lookup(si, ti, source_table, target_table)
    return (so.reshape(BATCH, SEQ, EMBED), to.reshape(BATCH, SEQ, EMBED))
